# Initial kernel scaffold; baseline (speedup 1.0000x reference)
import jax
import jax.numpy as jnp
from jax.experimental import pallas as pl


def kernel(reps, scope, label, W, b):
    raise NotImplementedError("write your pallas kernel here")


# speedup vs baseline: 21.8204x; 21.8204x over previous
import functools

import jax
import jax.numpy as jnp
from jax import lax
from jax.experimental import pallas as pl
from jax.experimental.pallas import tpu as pltpu
from jax.experimental.pallas import tpu_sc as plsc

N = 16384
B = 4096
HIDDEN = 768
C = 53
D = 64
R = 512
NW = 32
BPW = B // NW
WIN = 144
NEG = -1e30


def _mm_prefix_body(reps_ref, wt_ref, b_ref, out_ref, carry_ref):
    i = pl.program_id(0)

    @pl.when(i == 0)
    def _():
        carry_ref[...] = jnp.zeros_like(carry_ref)

    y = jnp.dot(reps_ref[...], wt_ref[...],
                preferred_element_type=jnp.float32) + b_ref[...]
    row = lax.broadcasted_iota(jnp.int32, (R, D), 0)
    sh = 1
    while sh < R:
        y = y + jnp.where(row >= sh, pltpu.roll(y, sh, 0), 0.0)
        sh *= 2
    y = y + carry_ref[...]
    out_ref[...] = y
    carry_ref[...] = y[R - 1:R, :]


def _mm_prefix(reps, wt, b_row):
    return pl.pallas_call(
        _mm_prefix_body,
        grid=(N // R,),
        in_specs=[
            pl.BlockSpec((R, HIDDEN), lambda i: (i, 0)),
            pl.BlockSpec((HIDDEN, D), lambda i: (0, 0)),
            pl.BlockSpec((1, D), lambda i: (0, 0)),
        ],
        out_specs=pl.BlockSpec((R, D), lambda i: (i, 0)),
        out_shape=jax.ShapeDtypeStruct((N, D), jnp.float32),
        scratch_shapes=[pltpu.VMEM((1, D), jnp.float32)],
        compiler_params=pltpu.CompilerParams(
            dimension_semantics=("arbitrary",)),
    )(reps, wt, b_row)


def _sc_bag_softmax_body(p_hbm, scope_hbm, out_hbm, idx_v, rows_v, out_v, sem):
    wid = lax.axis_index("s") * 2 + lax.axis_index("c")
    base = wid * BPW
    pltpu.sync_copy(scope_hbm.at[pl.ds(base, WIN)], idx_v)
    cp1 = pltpu.async_copy(p_hbm.at[idx_v.at[pl.ds(0, 128)]],
                           rows_v.at[pl.ds(0, 128)], sem)
    cp2 = pltpu.async_copy(p_hbm.at[idx_v.at[pl.ds(128, WIN - 128)]],
                           rows_v.at[pl.ds(128, WIN - 128)], sem)
    cp1.wait()
    cp2.wait()

    lane = lax.iota(jnp.int32, 16)
    mask3 = lane < (C - 48)

    def body(i, carry):
        win = idx_v[pl.ds(i, 16)]
        inv = 1.0 / jnp.full((16,), win[1] - win[0], jnp.int32).astype(jnp.float32)
        d = []
        for k in range(4):
            a = rows_v[i + 1, pl.ds(16 * k, 16)]
            s = rows_v[i, pl.ds(16 * k, 16)]
            d.append((a - s) * inv)
        d[3] = jnp.where(mask3, d[3], NEG)
        m = jnp.maximum(jnp.maximum(jnp.max(d[0]), jnp.max(d[1])),
                        jnp.maximum(jnp.max(d[2]), jnp.max(d[3])))
        e = [jnp.exp(x - m) for x in d]
        s_tot = (jnp.sum(e[0]) + jnp.sum(e[1])) + (jnp.sum(e[2]) + jnp.sum(e[3]))
        r = 1.0 / jnp.full((16,), s_tot, jnp.float32)
        for k in range(4):
            out_v[i, pl.ds(16 * k, 16)] = e[k] * r
        return carry

    lax.fori_loop(0, BPW, body, 0)
    pltpu.sync_copy(out_v, out_hbm.at[pl.ds(base, BPW)])


@functools.lru_cache(maxsize=1)
def _sc_bag_softmax():
    mesh = plsc.VectorSubcoreMesh(core_axis_name="c", subcore_axis_name="s")
    return pl.kernel(
        _sc_bag_softmax_body,
        mesh=mesh,
        out_type=jax.ShapeDtypeStruct((B, D), jnp.float32),
        scratch_types=[
            pltpu.VMEM((WIN,), jnp.int32),
            pltpu.VMEM((WIN, D), jnp.float32),
            pltpu.VMEM((BPW, D), jnp.float32),
            pltpu.SemaphoreType.DMA,
        ],
        compiler_params=pltpu.CompilerParams(needs_layout_passes=False,
                                             use_tc_tiling_on_sc=False),
    )


def kernel(reps, scope, label, W, b):
    del label
    wt = jnp.pad(W, ((0, D - C), (0, 0))).T
    b_row = jnp.pad(b, (0, D - C)).reshape(1, D)
    csum = _mm_prefix(reps, wt, b_row)
    prefix = jnp.concatenate(
        [jnp.zeros((1, D), jnp.float32), csum])
    scope_pad = jnp.pad(scope, (0, (NW - 1) * BPW + WIN - (B + 1)),
                        mode="edge")
    probs = _sc_bag_softmax()(prefix, scope_pad)
    return probs[:, :C]


# speedup vs baseline: 26.1724x; 1.1994x over previous
import functools

import jax
import jax.numpy as jnp
from jax import lax
from jax.experimental import pallas as pl
from jax.experimental.pallas import tpu as pltpu
from jax.experimental.pallas import tpu_sc as plsc

N = 16384
B = 4096
HIDDEN = 768
C = 53
D = 64
R = 512
NW = 32
BPW = B // NW
WIN = 144
NEG = -1e30


def _mm_prefix_body(reps_ref, wt_ref, b_ref, out_ref, carry_ref):
    i = pl.program_id(0)

    @pl.when(i == 0)
    def _():
        carry_ref[...] = jnp.zeros_like(carry_ref)

    y = jnp.dot(reps_ref[...], wt_ref[...],
                preferred_element_type=jnp.float32) + b_ref[...]
    row = lax.broadcasted_iota(jnp.int32, (R, D), 0)
    sh = 1
    while sh < R:
        y = y + jnp.where(row >= sh, pltpu.roll(y, sh, 0), 0.0)
        sh *= 2
    y = y + carry_ref[...]
    out_ref[...] = jnp.where(row >= 1, pltpu.roll(y, 1, 0), carry_ref[...])
    carry_ref[...] = y[R - 1:R, :]


def _mm_prefix(reps, wt, b_row):
    return pl.pallas_call(
        _mm_prefix_body,
        grid=(N // R + 1,),
        in_specs=[
            pl.BlockSpec((R, HIDDEN), lambda i: (jnp.minimum(i, N // R - 1), 0)),
            pl.BlockSpec((HIDDEN, D), lambda i: (0, 0)),
            pl.BlockSpec((1, D), lambda i: (0, 0)),
        ],
        out_specs=pl.BlockSpec((R, D), lambda i: (i, 0)),
        out_shape=jax.ShapeDtypeStruct((N + R, D), jnp.float32),
        scratch_shapes=[pltpu.VMEM((1, D), jnp.float32)],
        compiler_params=pltpu.CompilerParams(
            dimension_semantics=("arbitrary",)),
    )(reps, wt, b_row)


def _sc_bag_softmax_body(p_hbm, scope_hbm, out_hbm, idx_v, rows_v, out_v,
                         inv_v, sem):
    wid = lax.axis_index("s") * 2 + lax.axis_index("c")
    base = wid * BPW
    pltpu.sync_copy(scope_hbm.at[pl.ds(base, WIN)], idx_v)
    cp1 = pltpu.async_copy(p_hbm.at[idx_v.at[pl.ds(0, 128)]],
                           rows_v.at[pl.ds(0, 128)], sem)
    cp2 = pltpu.async_copy(p_hbm.at[idx_v.at[pl.ds(128, WIN - 128)]],
                           rows_v.at[pl.ds(128, WIN - 128)], sem)
    for k in range(BPW // 16):
        lo = idx_v[pl.ds(16 * k, 16)]
        hi = idx_v[pl.ds(16 * k + 1, 16)]
        inv_v[pl.ds(16 * k, 16)] = 1.0 / (hi - lo).astype(jnp.float32)
    cp1.wait()
    cp2.wait()

    lane = lax.iota(jnp.int32, 16)
    mask3 = lane < (C - 48)

    @plsc.parallel_loop(0, BPW, unroll=4)
    def body(i):
        inv = plsc.load_gather(inv_v, [jnp.full((16,), i, jnp.int32)])
        d = []
        for k in range(4):
            a = rows_v[i + 1, pl.ds(16 * k, 16)]
            s = rows_v[i, pl.ds(16 * k, 16)]
            d.append((a - s) * inv)
        d[3] = jnp.where(mask3, d[3], NEG)
        m = jnp.max(jnp.maximum(jnp.maximum(d[0], d[1]),
                                jnp.maximum(d[2], d[3])))
        e = [jnp.exp(x - m) for x in d]
        s_tot = jnp.sum((e[0] + e[1]) + (e[2] + e[3]))
        r = 1.0 / jnp.full((16,), s_tot, jnp.float32)
        for k in range(4):
            out_v[i, pl.ds(16 * k, 16)] = e[k] * r

    pltpu.sync_copy(out_v, out_hbm.at[pl.ds(base, BPW)])


@functools.lru_cache(maxsize=1)
def _sc_bag_softmax():
    mesh = plsc.VectorSubcoreMesh(core_axis_name="c", subcore_axis_name="s")
    return pl.kernel(
        _sc_bag_softmax_body,
        mesh=mesh,
        out_type=jax.ShapeDtypeStruct((B, D), jnp.float32),
        scratch_types=[
            pltpu.VMEM((WIN,), jnp.int32),
            pltpu.VMEM((WIN, D), jnp.float32),
            pltpu.VMEM((BPW, D), jnp.float32),
            pltpu.VMEM((BPW,), jnp.float32),
            pltpu.SemaphoreType.DMA,
        ],
        compiler_params=pltpu.CompilerParams(needs_layout_passes=False,
                                             use_tc_tiling_on_sc=False),
    )


def kernel(reps, scope, label, W, b):
    del label
    wt = jnp.pad(W, ((0, D - C), (0, 0))).T
    b_row = jnp.pad(b, (0, D - C)).reshape(1, D)
    prefix = _mm_prefix(reps, wt, b_row)
    scope_pad = jnp.pad(scope, (0, (NW - 1) * BPW + WIN - (B + 1)),
                        mode="edge")
    probs = _sc_bag_softmax()(prefix, scope_pad)
    return probs[:, :C]


# speedup vs baseline: 30.1061x; 1.1503x over previous
import functools

import jax
import jax.numpy as jnp
from jax import lax
from jax.experimental import pallas as pl
from jax.experimental.pallas import tpu as pltpu
from jax.experimental.pallas import tpu_sc as plsc

N = 16384
B = 4096
HIDDEN = 768
C = 53
D = 64
R = 1024
NW = 32
BPW = B // NW
WIN = 144
NEG = -1e30


def _mm_prefix_body(reps_ref, wt_ref, b_ref, out_ref, carry_ref):
    i = pl.program_id(0)

    @pl.when(i == 0)
    def _():
        carry_ref[...] = jnp.zeros_like(carry_ref)

    y = jnp.dot(reps_ref[...], wt_ref[...],
                preferred_element_type=jnp.float32) + b_ref[...]
    row = lax.broadcasted_iota(jnp.int32, (R, D), 0)
    sh = 1
    while sh < R:
        y = y + jnp.where(row >= sh, pltpu.roll(y, sh, 0), 0.0)
        sh *= 2
    y = y + carry_ref[...]
    out_ref[...] = jnp.where(row >= 1, pltpu.roll(y, 1, 0), carry_ref[...])
    carry_ref[...] = y[R - 1:R, :]


def _mm_prefix(reps, wt, b_row):
    return pl.pallas_call(
        _mm_prefix_body,
        grid=(N // R + 1,),
        in_specs=[
            pl.BlockSpec((R, HIDDEN), lambda i: (jnp.minimum(i, N // R - 1), 0)),
            pl.BlockSpec((HIDDEN, D), lambda i: (0, 0)),
            pl.BlockSpec((1, D), lambda i: (0, 0)),
        ],
        out_specs=pl.BlockSpec((R, D), lambda i: (i, 0)),
        out_shape=jax.ShapeDtypeStruct((N + R, D), jnp.float32),
        scratch_shapes=[pltpu.VMEM((1, D), jnp.float32)],
        compiler_params=pltpu.CompilerParams(
            dimension_semantics=("arbitrary",)),
    )(reps, wt, b_row)


def _sc_bag_softmax_body(p_hbm, scope_hbm, out_hbm, idx_v, rows_v, out_v,
                         inv_v, sem):
    wid = lax.axis_index("s") * 2 + lax.axis_index("c")
    base = wid * BPW
    pltpu.sync_copy(scope_hbm.at[pl.ds(base, WIN)], idx_v)
    cp1 = pltpu.async_copy(p_hbm.at[idx_v.at[pl.ds(0, 128)]],
                           rows_v.at[pl.ds(0, 128)], sem)
    cp2 = pltpu.async_copy(p_hbm.at[idx_v.at[pl.ds(128, WIN - 128)]],
                           rows_v.at[pl.ds(128, WIN - 128)], sem)
    for k in range(BPW // 16):
        lo = idx_v[pl.ds(16 * k, 16)]
        hi = idx_v[pl.ds(16 * k + 1, 16)]
        inv_v[pl.ds(16 * k, 16)] = 1.0 / (hi - lo).astype(jnp.float32)
    cp1.wait()
    cp2.wait()

    lane = lax.iota(jnp.int32, 16)
    mask3 = lane < (C - 48)

    @plsc.parallel_loop(0, BPW, unroll=4)
    def body(i):
        inv = plsc.load_gather(inv_v, [jnp.full((16,), i, jnp.int32)])
        d = []
        for k in range(4):
            a = rows_v[i + 1, pl.ds(16 * k, 16)]
            s = rows_v[i, pl.ds(16 * k, 16)]
            d.append((a - s) * inv)
        d[3] = jnp.where(mask3, d[3], NEG)
        m = jnp.max(jnp.maximum(jnp.maximum(d[0], d[1]),
                                jnp.maximum(d[2], d[3])))
        e = [jnp.exp(x - m) for x in d]
        s_tot = jnp.sum((e[0] + e[1]) + (e[2] + e[3]))
        r = 1.0 / jnp.full((16,), s_tot, jnp.float32)
        for k in range(4):
            out_v[i, pl.ds(16 * k, 16)] = e[k] * r

    pltpu.sync_copy(out_v, out_hbm.at[pl.ds(base, BPW)])


@functools.lru_cache(maxsize=1)
def _sc_bag_softmax():
    mesh = plsc.VectorSubcoreMesh(core_axis_name="c", subcore_axis_name="s")
    return pl.kernel(
        _sc_bag_softmax_body,
        mesh=mesh,
        out_type=jax.ShapeDtypeStruct((B, D), jnp.float32),
        scratch_types=[
            pltpu.VMEM((WIN,), jnp.int32),
            pltpu.VMEM((WIN, D), jnp.float32),
            pltpu.VMEM((BPW, D), jnp.float32),
            pltpu.VMEM((BPW,), jnp.float32),
            pltpu.SemaphoreType.DMA,
        ],
        compiler_params=pltpu.CompilerParams(needs_layout_passes=False,
                                             use_tc_tiling_on_sc=False),
    )


def kernel(reps, scope, label, W, b):
    del label
    wt = jnp.pad(W, ((0, D - C), (0, 0))).T
    b_row = jnp.pad(b, (0, D - C)).reshape(1, D)
    prefix = _mm_prefix(reps, wt, b_row)
    scope_pad = jnp.pad(scope, (0, (NW - 1) * BPW + WIN - (B + 1)),
                        mode="edge")
    probs = _sc_bag_softmax()(prefix, scope_pad)
    return probs[:, :C]


# speedup vs baseline: 31.4182x; 1.0436x over previous
import functools

import jax
import jax.numpy as jnp
from jax import lax
from jax.experimental import pallas as pl
from jax.experimental.pallas import tpu as pltpu
from jax.experimental.pallas import tpu_sc as plsc

N = 16384
B = 4096
HIDDEN = 768
C = 53
D = 64
R = 2048
NW = 32
BPW = B // NW
WIN = 144
NEG = -1e30


def _mm_prefix_body(reps_ref, wt_ref, b_ref, out_ref, carry_ref):
    i = pl.program_id(0)

    @pl.when(i == 0)
    def _():
        carry_ref[...] = jnp.zeros_like(carry_ref)

    y = jnp.dot(reps_ref[...], wt_ref[...],
                preferred_element_type=jnp.float32) + b_ref[...]
    row = lax.broadcasted_iota(jnp.int32, (R, D), 0)
    sh = 1
    while sh < R:
        y = y + jnp.where(row >= sh, pltpu.roll(y, sh, 0), 0.0)
        sh *= 2
    y = y + carry_ref[...]
    out_ref[...] = jnp.where(row >= 1, pltpu.roll(y, 1, 0), carry_ref[...])
    carry_ref[...] = y[R - 1:R, :]


def _mm_prefix(reps, wt, b_row):
    return pl.pallas_call(
        _mm_prefix_body,
        grid=(N // R + 1,),
        in_specs=[
            pl.BlockSpec((R, HIDDEN), lambda i: (jnp.minimum(i, N // R - 1), 0)),
            pl.BlockSpec((HIDDEN, D), lambda i: (0, 0)),
            pl.BlockSpec((1, D), lambda i: (0, 0)),
        ],
        out_specs=pl.BlockSpec((R, D), lambda i: (i, 0)),
        out_shape=jax.ShapeDtypeStruct((N + R, D), jnp.float32),
        scratch_shapes=[pltpu.VMEM((1, D), jnp.float32)],
        compiler_params=pltpu.CompilerParams(
            dimension_semantics=("arbitrary",)),
    )(reps, wt, b_row)


def _sc_bag_softmax_body(p_hbm, scope_hbm, out_hbm, idx_v, rows_v, out_v,
                         inv_v, sem):
    wid = lax.axis_index("s") * 2 + lax.axis_index("c")
    base = wid * BPW
    pltpu.sync_copy(scope_hbm.at[pl.ds(base, WIN)], idx_v)
    cp1 = pltpu.async_copy(p_hbm.at[idx_v.at[pl.ds(0, 128)]],
                           rows_v.at[pl.ds(0, 128)], sem)
    cp2 = pltpu.async_copy(p_hbm.at[idx_v.at[pl.ds(128, WIN - 128)]],
                           rows_v.at[pl.ds(128, WIN - 128)], sem)
    for k in range(BPW // 16):
        lo = idx_v[pl.ds(16 * k, 16)]
        hi = idx_v[pl.ds(16 * k + 1, 16)]
        inv_v[pl.ds(16 * k, 16)] = 1.0 / (hi - lo).astype(jnp.float32)
    cp1.wait()
    cp2.wait()

    lane = lax.iota(jnp.int32, 16)
    mask3 = lane < (C - 48)

    @plsc.parallel_loop(0, BPW, unroll=4)
    def body(i):
        inv = plsc.load_gather(inv_v, [jnp.full((16,), i, jnp.int32)])
        d = []
        for k in range(4):
            a = rows_v[i + 1, pl.ds(16 * k, 16)]
            s = rows_v[i, pl.ds(16 * k, 16)]
            d.append((a - s) * inv)
        d[3] = jnp.where(mask3, d[3], NEG)
        m = jnp.max(jnp.maximum(jnp.maximum(d[0], d[1]),
                                jnp.maximum(d[2], d[3])))
        e = [jnp.exp(x - m) for x in d]
        s_tot = jnp.sum((e[0] + e[1]) + (e[2] + e[3]))
        r = 1.0 / jnp.full((16,), s_tot, jnp.float32)
        for k in range(4):
            out_v[i, pl.ds(16 * k, 16)] = e[k] * r

    pltpu.sync_copy(out_v, out_hbm.at[pl.ds(base, BPW)])


@functools.lru_cache(maxsize=1)
def _sc_bag_softmax():
    mesh = plsc.VectorSubcoreMesh(core_axis_name="c", subcore_axis_name="s")
    return pl.kernel(
        _sc_bag_softmax_body,
        mesh=mesh,
        out_type=jax.ShapeDtypeStruct((B, D), jnp.float32),
        scratch_types=[
            pltpu.VMEM((WIN,), jnp.int32),
            pltpu.VMEM((WIN, D), jnp.float32),
            pltpu.VMEM((BPW, D), jnp.float32),
            pltpu.VMEM((BPW,), jnp.float32),
            pltpu.SemaphoreType.DMA,
        ],
        compiler_params=pltpu.CompilerParams(needs_layout_passes=False,
                                             use_tc_tiling_on_sc=False),
    )


def kernel(reps, scope, label, W, b):
    del label
    wt = jnp.pad(W, ((0, D - C), (0, 0))).T
    b_row = jnp.pad(b, (0, D - C)).reshape(1, D)
    prefix = _mm_prefix(reps, wt, b_row)
    scope_pad = jnp.pad(scope, (0, (NW - 1) * BPW + WIN - (B + 1)),
                        mode="edge")
    probs = _sc_bag_softmax()(prefix, scope_pad)
    return probs[:, :C]


# speedup vs baseline: 33.0956x; 1.0534x over previous
import functools

import jax
import jax.numpy as jnp
from jax import lax
from jax.experimental import pallas as pl
from jax.experimental.pallas import tpu as pltpu
from jax.experimental.pallas import tpu_sc as plsc

N = 16384
B = 4096
HIDDEN = 768
C = 53
D = 64
R = 2048
NW = 32
BPW = B // NW
WIN = 144
NEG = -1e30


def _mm_prefix_body(reps_ref, wt_ref, b_ref, out_ref, carry_ref):
    i = pl.program_id(0)

    @pl.when(i == 0)
    def _():
        carry_ref[...] = jnp.zeros_like(carry_ref)

    y = jnp.dot(reps_ref[...], wt_ref[...],
                preferred_element_type=jnp.float32) + b_ref[...]
    row = lax.broadcasted_iota(jnp.int32, (R, D), 0)
    sh = 1
    while sh < R:
        y = y + jnp.where(row >= sh, pltpu.roll(y, sh, 0), 0.0)
        sh *= 2
    y = y + carry_ref[...]
    out_ref[...] = y
    carry_ref[...] = y[R - 1:R, :]


def _mm_prefix(reps, wt, b_row):
    return pl.pallas_call(
        _mm_prefix_body,
        grid=(N // R,),
        in_specs=[
            pl.BlockSpec((R, HIDDEN), lambda i: (i, 0)),
            pl.BlockSpec((HIDDEN, D), lambda i: (0, 0)),
            pl.BlockSpec((1, D), lambda i: (0, 0)),
        ],
        out_specs=pl.BlockSpec((R, D), lambda i: (i, 0)),
        out_shape=jax.ShapeDtypeStruct((N, D), jnp.float32),
        scratch_shapes=[pltpu.VMEM((1, D), jnp.float32)],
        compiler_params=pltpu.CompilerParams(
            dimension_semantics=("arbitrary",)),
    )(reps, wt, b_row)


def _sc_bag_softmax_body(p_hbm, scope_hbm, out_hbm, idx_v, rows_v, out_v,
                         inv_v, sem):
    wid = lax.axis_index("s") * 2 + lax.axis_index("c")
    base = wid * BPW
    pltpu.sync_copy(scope_hbm.at[pl.ds(base, WIN)], idx_v)
    cp1 = pltpu.async_copy(p_hbm.at[idx_v.at[pl.ds(0, 128)]],
                           rows_v.at[pl.ds(0, 128)], sem)
    cp2 = pltpu.async_copy(p_hbm.at[idx_v.at[pl.ds(128, WIN - 128)]],
                           rows_v.at[pl.ds(128, WIN - 128)], sem)
    lane = lax.iota(jnp.int32, 16)
    for k in range(BPW // 16):
        lo = idx_v[pl.ds(16 * k, 16)]
        hi = idx_v[pl.ds(16 * k + 1, 16)]
        inv_v[pl.ds(16 * k, 16)] = 1.0 / (hi - lo).astype(jnp.float32)

    @pl.when(wid == 0)
    def _():
        lo = idx_v[pl.ds(0, 16)]
        hi = idx_v[pl.ds(1, 16)]
        cnt = (hi - lo).astype(jnp.float32) + jnp.where(lane == 0, 1.0, 0.0)
        inv_v[pl.ds(0, 16)] = 1.0 / cnt

    cp1.wait()
    cp2.wait()

    @pl.when(wid == 0)
    def _():
        for k in range(4):
            rows_v[0, pl.ds(16 * k, 16)] = jnp.zeros((16,), jnp.float32)

    mask3 = lane < (C - 48)

    @plsc.parallel_loop(0, BPW, unroll=4)
    def body(i):
        inv = plsc.load_gather(inv_v, [jnp.full((16,), i, jnp.int32)])
        d = []
        for k in range(4):
            a = rows_v[i + 1, pl.ds(16 * k, 16)]
            s = rows_v[i, pl.ds(16 * k, 16)]
            d.append((a - s) * inv)
        d[3] = jnp.where(mask3, d[3], NEG)
        m = jnp.max(jnp.maximum(jnp.maximum(d[0], d[1]),
                                jnp.maximum(d[2], d[3])))
        e = [jnp.exp(x - m) for x in d]
        s_tot = jnp.sum((e[0] + e[1]) + (e[2] + e[3]))
        r = 1.0 / jnp.full((16,), s_tot, jnp.float32)
        for k in range(4):
            out_v[i, pl.ds(16 * k, 16)] = e[k] * r

    pltpu.sync_copy(out_v, out_hbm.at[pl.ds(base, BPW)])


@functools.lru_cache(maxsize=1)
def _sc_bag_softmax():
    mesh = plsc.VectorSubcoreMesh(core_axis_name="c", subcore_axis_name="s")
    return pl.kernel(
        _sc_bag_softmax_body,
        mesh=mesh,
        out_type=jax.ShapeDtypeStruct((B, D), jnp.float32),
        scratch_types=[
            pltpu.VMEM((WIN,), jnp.int32),
            pltpu.VMEM((WIN, D), jnp.float32),
            pltpu.VMEM((BPW, D), jnp.float32),
            pltpu.VMEM((BPW,), jnp.float32),
            pltpu.SemaphoreType.DMA,
        ],
        compiler_params=pltpu.CompilerParams(needs_layout_passes=False,
                                             use_tc_tiling_on_sc=False),
    )


def kernel(reps, scope, label, W, b):
    del label
    wt = jnp.pad(W, ((0, D - C), (0, 0))).T
    b_row = jnp.pad(b, (0, D - C)).reshape(1, D)
    prefix = _mm_prefix(reps, wt, b_row)
    idx = jnp.maximum(scope - 1, 0)
    scope_pad = jnp.pad(idx, (0, (NW - 1) * BPW + WIN - (B + 1)),
                        mode="edge")
    probs = _sc_bag_softmax()(prefix, scope_pad)
    return probs[:, :C]
